# Initial kernel scaffold; baseline (speedup 1.0000x reference)
import functools

import jax
import jax.numpy as jnp
from jax import lax
from jax.experimental import pallas as pl
from jax.experimental.pallas import tpu as pltpu
from jax.experimental.pallas import tpu_sc as plsc

N_NODES = 10000
N_EDGES = 320000
D = 128

NW = 32
CHUNK = 80
NCHUNK = N_EDGES // NW // CHUNK
N_PAD = 10016
ROWS_PER_TILE = N_PAD // 16


def _sc_agg_kernel(x_hbm, src_hbm, dst_hbm, zblk_hbm, zrow_hbm,
                   agg_out, deg_out,
                   src_v, dst_v, rows_v, deg_v, agg_s, sem):
    c = lax.axis_index("c")
    s = lax.axis_index("s")
    wid = c * 16 + s

    pltpu.sync_copy(src_hbm.at[wid], src_v)
    pltpu.sync_copy(dst_hbm.at[wid], dst_v)
    pltpu.sync_copy(zrow_hbm, deg_v)
    base = s * ROWS_PER_TILE
    pltpu.sync_copy(zblk_hbm, agg_s.at[pl.ds(base, ROWS_PER_TILE)])
    plsc.subcore_barrier()

    ones = jnp.full((16,), 1.0, jnp.float32)

    def chunk_body(j, carry):
        pltpu.async_copy(x_hbm.at[src_v.at[j]], rows_v, sem).wait()
        pltpu.sync_copy(rows_v, agg_s.at[dst_v.at[j]], add=True)
        def deg_body(t, carry2):
            idx = dst_v[j, pl.ds(t * 16, 16)]
            plsc.addupdate_scatter(deg_v, [idx], ones)
            return carry2
        lax.fori_loop(0, CHUNK // 16, deg_body, 0)
        return carry

    lax.fori_loop(0, NCHUNK, chunk_body, 0)
    plsc.subcore_barrier()

    pltpu.sync_copy(agg_s.at[pl.ds(base, ROWS_PER_TILE)],
                    agg_out.at[c, pl.ds(base, ROWS_PER_TILE)])
    pltpu.sync_copy(deg_v, deg_out.at[wid])


def _sc_agg(x, src_r, dst_r):
    zblk = jnp.zeros((ROWS_PER_TILE, D), jnp.float32)
    zrow = jnp.zeros((N_PAD,), jnp.float32)
    mesh = plsc.VectorSubcoreMesh(core_axis_name="c", subcore_axis_name="s")
    fn = functools.partial(
        pl.kernel,
        mesh=mesh,
        out_type=[
            jax.ShapeDtypeStruct((2, N_PAD, D), jnp.float32),
            jax.ShapeDtypeStruct((NW, N_PAD), jnp.float32),
        ],
        scratch_types=[
            pltpu.VMEM((NCHUNK, CHUNK), jnp.int32),
            pltpu.VMEM((NCHUNK, CHUNK), jnp.int32),
            pltpu.VMEM((CHUNK, D), jnp.float32),
            pltpu.VMEM((N_PAD,), jnp.float32),
            pltpu.VMEM_SHARED((N_PAD, D), jnp.float32),
            pltpu.SemaphoreType.DMA,
        ],
    )
    return fn(_sc_agg_kernel)(x, src_r, dst_r, zblk, zrow)


def _tc_combine_kernel(x_ref, agg_ref, deg_ref, w_ref, o_ref):
    deg = jnp.sum(deg_ref[...], axis=0)
    den = jnp.maximum(deg, 1.0)
    agg = agg_ref[0] + agg_ref[1]
    nb = agg / den[:, None]
    w = w_ref[...]
    out = lax.dot_general(x_ref[...], w[:, :D], (((1,), (1,)), ((), ())),
                          preferred_element_type=jnp.float32)
    out = out + lax.dot_general(nb, w[:, D:], (((1,), (1,)), ((), ())),
                                preferred_element_type=jnp.float32)
    o_ref[...] = out


def _tc_combine(x, agg_p, deg_p, w):
    blk = 1000
    grid = (N_NODES // blk,)
    return pl.pallas_call(
        _tc_combine_kernel,
        grid=grid,
        in_specs=[
            pl.BlockSpec((blk, D), lambda i: (i, 0)),
            pl.BlockSpec((2, blk, D), lambda i: (0, i, 0)),
            pl.BlockSpec((NW, blk), lambda i: (0, i)),
            pl.BlockSpec((D, 2 * D), lambda i: (0, 0)),
        ],
        out_specs=pl.BlockSpec((blk, D), lambda i: (i, 0)),
        out_shape=jax.ShapeDtypeStruct((N_NODES, D), jnp.float32),
    )(x, agg_p, deg_p, w)


@jax.jit
def kernel(X, adj, W):
    src = adj[0].astype(jnp.int32).reshape(NW, NCHUNK, CHUNK)
    dst = adj[1].astype(jnp.int32).reshape(NW, NCHUNK, CHUNK)
    agg_p, deg_p = _sc_agg(X, src, dst)
    agg_p = agg_p[:, :N_NODES, :]
    deg_p = deg_p[:, :N_NODES]
    return _tc_combine(X, agg_p, deg_p, W)


# speedup vs baseline: 8.1555x; 8.1555x over previous
import functools

import jax
import jax.numpy as jnp
from jax import lax
from jax.experimental import pallas as pl
from jax.experimental.pallas import tpu as pltpu
from jax.experimental.pallas import tpu_sc as plsc

N_NODES = 10000
N_EDGES = 320000
D = 128

NW = 32
CHUNK = 80
NCHUNK = N_EDGES // NW // CHUNK
N_PAD = 10112
ROWS_PER_TILE = N_PAD // 16
DEG_N = N_NODES


def _sc_agg_kernel(x_hbm, comb_hbm,
                   agg_out, deg_out,
                   comb_v, src_c, dst_c, doff_c, rows_v, ones_v, deg_v,
                   agg_s, deg_s, sem):
    c = lax.axis_index("c")
    s = lax.axis_index("s")
    wid = c * 16 + s

    pltpu.sync_copy(comb_hbm.at[wid], comb_v)

    ones = jnp.full((16,), 1.0, jnp.float32)
    zeros = jnp.zeros((16,), jnp.float32)
    for t in range(CHUNK // 16):
        ones_v[pl.ds(t * 16, 16)] = ones

    def zrows_body(i, carry):
        r = i // (D // 16)
        t = i % (D // 16)
        rows_v[r, pl.ds(t * 16, 16)] = zeros
        return carry
    lax.fori_loop(0, CHUNK * (D // 16), zrows_body, 0)
    base = s * ROWS_PER_TILE
    for b in range(ROWS_PER_TILE // CHUNK):
        pltpu.sync_copy(rows_v, agg_s.at[pl.ds(base + b * CHUNK, CHUNK)])
    rem = ROWS_PER_TILE % CHUNK
    if rem:
        pltpu.sync_copy(
            rows_v.at[pl.ds(0, rem)],
            agg_s.at[pl.ds(base + (ROWS_PER_TILE // CHUNK) * CHUNK, rem)])

    def zdeg_body(t, carry):
        deg_v[pl.ds(t * 16, 16)] = zeros
        return carry
    lax.fori_loop(0, DEG_N // 16, zdeg_body, 0)
    pltpu.sync_copy(deg_v, deg_s.at[pl.ds(s * DEG_N, DEG_N)])
    plsc.subcore_barrier()

    soff = s * DEG_N

    def chunk_body(j, carry):
        for t in range(CHUNK // 16):
            cv = comb_v[j, pl.ds(t * 16, 16)]
            sv = jnp.bitwise_and(cv, 0xFFFF)
            dv = lax.shift_right_logical(cv, 16)
            src_c[pl.ds(t * 16, 16)] = sv
            dst_c[pl.ds(t * 16, 16)] = dv
            doff_c[pl.ds(t * 16, 16)] = dv + soff
        pltpu.async_copy(x_hbm.at[src_c], rows_v, sem).wait()
        pltpu.sync_copy(rows_v, agg_s.at[dst_c], add=True)
        pltpu.sync_copy(ones_v, deg_s.at[doff_c], add=True)
        return carry

    lax.fori_loop(0, NCHUNK, chunk_body, 0)
    plsc.subcore_barrier()

    pltpu.sync_copy(agg_s.at[pl.ds(base, ROWS_PER_TILE)],
                    agg_out.at[c, pl.ds(base, ROWS_PER_TILE)])
    pltpu.sync_copy(deg_s.at[pl.ds(s * DEG_N, DEG_N)], deg_v)
    pltpu.sync_copy(deg_v, deg_out.at[pl.ds(wid * DEG_N, DEG_N)])


def _sc_agg(x, comb_r):
    mesh = plsc.VectorSubcoreMesh(core_axis_name="c", subcore_axis_name="s")
    fn = functools.partial(
        pl.kernel,
        mesh=mesh,
        out_type=[
            jax.ShapeDtypeStruct((2, N_PAD, D), jnp.float32),
            jax.ShapeDtypeStruct((NW * DEG_N,), jnp.float32),
        ],
        scratch_types=[
            pltpu.VMEM((NCHUNK, CHUNK), jnp.int32),
            pltpu.VMEM((CHUNK,), jnp.int32),
            pltpu.VMEM((CHUNK,), jnp.int32),
            pltpu.VMEM((CHUNK,), jnp.int32),
            pltpu.VMEM((CHUNK, D), jnp.float32),
            pltpu.VMEM((CHUNK,), jnp.float32),
            pltpu.VMEM((DEG_N,), jnp.float32),
            pltpu.VMEM_SHARED((N_PAD, D), jnp.float32),
            pltpu.VMEM_SHARED((16 * DEG_N,), jnp.float32),
            pltpu.SemaphoreType.DMA,
        ],
    )
    return fn(_sc_agg_kernel)(x, comb_r)


def _tc_combine_kernel(x_ref, agg_ref, deg_ref, w_ref, o_ref):
    deg = jnp.sum(deg_ref[...], axis=1)
    den = jnp.maximum(deg, 1.0)
    agg = agg_ref[0] + agg_ref[1]
    nb = agg / den[:, None]
    w = w_ref[...]
    out = lax.dot_general(x_ref[...], w[:, :D], (((1,), (1,)), ((), ())),
                          preferred_element_type=jnp.float32)
    out = out + lax.dot_general(nb, w[:, D:], (((1,), (1,)), ((), ())),
                                preferred_element_type=jnp.float32)
    o_ref[...] = out


def _tc_combine(x, agg_p, deg_p, w):
    blk = 1000
    grid = (N_NODES // blk,)
    return pl.pallas_call(
        _tc_combine_kernel,
        grid=grid,
        in_specs=[
            pl.BlockSpec((blk, D), lambda i: (i, 0)),
            pl.BlockSpec((2, blk, D), lambda i: (0, i, 0)),
            pl.BlockSpec((blk, NW), lambda i: (i, 0)),
            pl.BlockSpec((D, 2 * D), lambda i: (0, 0)),
        ],
        out_specs=pl.BlockSpec((blk, D), lambda i: (i, 0)),
        out_shape=jax.ShapeDtypeStruct((N_NODES, D), jnp.float32),
    )(x, agg_p, deg_p, w)


@jax.jit
def kernel(X, adj, W):
    src = adj[0].astype(jnp.int32)
    dst = adj[1].astype(jnp.int32)
    comb = (src + (dst << 16)).reshape(NW, NCHUNK, CHUNK)
    agg_p, deg_p = _sc_agg(X, comb)
    agg_p = agg_p[:, :N_NODES, :]
    deg_p = deg_p.reshape(NW, DEG_N).T
    return _tc_combine(X, agg_p, deg_p, W)


# speedup vs baseline: 12.4760x; 1.5298x over previous
import functools

import jax
import jax.numpy as jnp
from jax import lax
from jax.experimental import pallas as pl
from jax.experimental.pallas import tpu as pltpu
from jax.experimental.pallas import tpu_sc as plsc

N_NODES = 10000
N_EDGES = 320000
D = 128

NW = 32
CHUNK = 80
NCHUNK = N_EDGES // NW // CHUNK
N_PAD = 10112
ROWS_PER_TILE = N_PAD // 16
DEG_N = N_NODES
ZBUF = 640


def _sc_agg_kernel(x_hbm, comb_hbm,
                   agg_out, deg_out,
                   comb_v, src_a, dst_a, doff_a, src_b, dst_b, doff_b,
                   rows_a, rows_b, ones_v, zbuf,
                   agg_s, deg_s, sem_a, sem_b):
    c = lax.axis_index("c")
    s = lax.axis_index("s")
    wid = c * 16 + s
    soff = s * DEG_N

    pltpu.sync_copy(comb_hbm.at[wid], comb_v)

    ones = jnp.full((16,), 1.0, jnp.float32)
    zeros = jnp.zeros((16,), jnp.float32)
    for t in range(CHUNK // 16):
        ones_v[pl.ds(t * 16, 16)] = ones

    def unpack(j, src_c, dst_c, doff_c):
        for t in range(CHUNK // 16):
            cv = comb_v[j, pl.ds(t * 16, 16)]
            sv = jnp.bitwise_and(cv, 0xFFFF)
            dv = lax.shift_right_logical(cv, 16)
            src_c[pl.ds(t * 16, 16)] = sv
            dst_c[pl.ds(t * 16, 16)] = dv
            doff_c[pl.ds(t * 16, 16)] = dv + soff

    def gather(src_c, rows_v, sem):
        pltpu.async_copy(x_hbm.at[src_c], rows_v, sem)

    def drain(src_c, rows_v, sem):
        pltpu.make_async_copy(x_hbm.at[src_c], rows_v, sem).wait()

    def scatter(dst_c, doff_c, rows_v):
        pltpu.sync_copy(rows_v, agg_s.at[dst_c], add=True)
        pltpu.sync_copy(ones_v, deg_s.at[doff_c], add=True)

    unpack(0, src_a, dst_a, doff_a)
    gather(src_a, rows_a, sem_a)

    def zrows_body(i, carry):
        r = i // (D // 16)
        t = i % (D // 16)
        rows_b[r, pl.ds(t * 16, 16)] = zeros
        return carry
    lax.fori_loop(0, CHUNK * (D // 16), zrows_body, 0)
    base = s * ROWS_PER_TILE
    for b in range(ROWS_PER_TILE // CHUNK):
        pltpu.sync_copy(rows_b, agg_s.at[pl.ds(base + b * CHUNK, CHUNK)])
    rem = ROWS_PER_TILE % CHUNK
    if rem:
        pltpu.sync_copy(
            rows_b.at[pl.ds(0, rem)],
            agg_s.at[pl.ds(base + (ROWS_PER_TILE // CHUNK) * CHUNK, rem)])

    def zbuf_body(t, carry):
        zbuf[pl.ds(t * 16, 16)] = zeros
        return carry
    lax.fori_loop(0, ZBUF // 16, zbuf_body, 0)
    for i in range(DEG_N // ZBUF):
        pltpu.sync_copy(zbuf, deg_s.at[pl.ds(soff + i * ZBUF, ZBUF)])
    drem = DEG_N % ZBUF
    if drem:
        pltpu.sync_copy(zbuf.at[pl.ds(0, drem)],
                        deg_s.at[pl.ds(soff + (DEG_N // ZBUF) * ZBUF, drem)])
    plsc.subcore_barrier()

    def pair_body(k, carry):
        j1 = 2 * k + 1
        unpack(j1, src_b, dst_b, doff_b)
        gather(src_b, rows_b, sem_b)
        drain(src_a, rows_a, sem_a)
        scatter(dst_a, doff_a, rows_a)
        unpack(2 * k + 2, src_a, dst_a, doff_a)
        gather(src_a, rows_a, sem_a)
        drain(src_b, rows_b, sem_b)
        scatter(dst_b, doff_b, rows_b)
        return carry

    lax.fori_loop(0, (NCHUNK - 1) // 2, pair_body, 0)
    drain(src_a, rows_a, sem_a)
    scatter(dst_a, doff_a, rows_a)
    plsc.subcore_barrier()

    pltpu.sync_copy(agg_s.at[pl.ds(base, ROWS_PER_TILE)],
                    agg_out.at[c, pl.ds(base, ROWS_PER_TILE)])
    for i in range(DEG_N // ZBUF):
        pltpu.sync_copy(deg_s.at[pl.ds(soff + i * ZBUF, ZBUF)], zbuf)
        pltpu.sync_copy(zbuf, deg_out.at[pl.ds(wid * DEG_N + i * ZBUF, ZBUF)])
    if DEG_N % ZBUF:
        i = DEG_N // ZBUF
        drem = DEG_N % ZBUF
        pltpu.sync_copy(deg_s.at[pl.ds(soff + i * ZBUF, drem)],
                        zbuf.at[pl.ds(0, drem)])
        pltpu.sync_copy(zbuf.at[pl.ds(0, drem)],
                        deg_out.at[pl.ds(wid * DEG_N + i * ZBUF, drem)])


def _sc_agg(x, comb_r):
    mesh = plsc.VectorSubcoreMesh(core_axis_name="c", subcore_axis_name="s")
    fn = functools.partial(
        pl.kernel,
        mesh=mesh,
        out_type=[
            jax.ShapeDtypeStruct((2, N_PAD, D), jnp.float32),
            jax.ShapeDtypeStruct((NW * DEG_N,), jnp.float32),
        ],
        scratch_types=[
            pltpu.VMEM((NCHUNK, CHUNK), jnp.int32),
            pltpu.VMEM((CHUNK,), jnp.int32),
            pltpu.VMEM((CHUNK,), jnp.int32),
            pltpu.VMEM((CHUNK,), jnp.int32),
            pltpu.VMEM((CHUNK,), jnp.int32),
            pltpu.VMEM((CHUNK,), jnp.int32),
            pltpu.VMEM((CHUNK,), jnp.int32),
            pltpu.VMEM((CHUNK, D), jnp.float32),
            pltpu.VMEM((CHUNK, D), jnp.float32),
            pltpu.VMEM((CHUNK,), jnp.float32),
            pltpu.VMEM((ZBUF,), jnp.float32),
            pltpu.VMEM_SHARED((N_PAD, D), jnp.float32),
            pltpu.VMEM_SHARED((16 * DEG_N,), jnp.float32),
            pltpu.SemaphoreType.DMA,
            pltpu.SemaphoreType.DMA,
        ],
    )
    return fn(_sc_agg_kernel)(x, comb_r)


def _tc_combine_kernel(x_ref, agg_ref, deg_ref, w_ref, o_ref):
    deg = jnp.sum(deg_ref[...], axis=1)
    den = jnp.maximum(deg, 1.0)
    agg = agg_ref[0] + agg_ref[1]
    nb = agg / den[:, None]
    w = w_ref[...]
    out = lax.dot_general(x_ref[...], w[:, :D], (((1,), (1,)), ((), ())),
                          preferred_element_type=jnp.float32)
    out = out + lax.dot_general(nb, w[:, D:], (((1,), (1,)), ((), ())),
                                preferred_element_type=jnp.float32)
    o_ref[...] = out


def _tc_combine(x, agg_p, deg_p, w):
    blk = 1000
    grid = (N_NODES // blk,)
    return pl.pallas_call(
        _tc_combine_kernel,
        grid=grid,
        in_specs=[
            pl.BlockSpec((blk, D), lambda i: (i, 0)),
            pl.BlockSpec((2, blk, D), lambda i: (0, i, 0)),
            pl.BlockSpec((blk, NW), lambda i: (i, 0)),
            pl.BlockSpec((D, 2 * D), lambda i: (0, 0)),
        ],
        out_specs=pl.BlockSpec((blk, D), lambda i: (i, 0)),
        out_shape=jax.ShapeDtypeStruct((N_NODES, D), jnp.float32),
    )(x, agg_p, deg_p, w)


@jax.jit
def kernel(X, adj, W):
    src = adj[0].astype(jnp.int32)
    dst = adj[1].astype(jnp.int32)
    comb = (src + (dst << 16)).reshape(NW, NCHUNK, CHUNK)
    agg_p, deg_p = _sc_agg(X, comb)
    agg_p = agg_p[:, :N_NODES, :]
    deg_p = deg_p.reshape(NW, DEG_N).T
    return _tc_combine(X, agg_p, deg_p, W)


# speedup vs baseline: 12.6601x; 1.0148x over previous
import functools

import jax
import jax.numpy as jnp
from jax import lax
from jax.experimental import pallas as pl
from jax.experimental.pallas import tpu as pltpu
from jax.experimental.pallas import tpu_sc as plsc

N_NODES = 10000
N_EDGES = 320000
D = 128

NW = 32
CHUNK = 80
NCHUNK = N_EDGES // NW // CHUNK
N_PAD = 10112
ROWS_PER_TILE = N_PAD // 16
DEG_N = N_NODES
ZBUF = 640


def _sc_agg_kernel(x_hbm, comb_hbm,
                   agg_out, deg_out,
                   comb_v, src_a, dst_a, doff_a, src_b, dst_b, doff_b,
                   rows_a, rows_b, ones_v, zbuf,
                   agg_s, deg_s, sem_a, sem_b, sem_da, sem_db):
    c = lax.axis_index("c")
    s = lax.axis_index("s")
    wid = c * 16 + s
    soff = s * DEG_N

    pltpu.sync_copy(comb_hbm.at[wid], comb_v)

    ones = jnp.full((16,), 1.0, jnp.float32)
    zeros = jnp.zeros((16,), jnp.float32)
    for t in range(CHUNK // 16):
        ones_v[pl.ds(t * 16, 16)] = ones

    def unpack(j, src_c, dst_c, doff_c):
        for t in range(CHUNK // 16):
            cv = comb_v[j, pl.ds(t * 16, 16)]
            sv = jnp.bitwise_and(cv, 0xFFFF)
            dv = lax.shift_right_logical(cv, 16)
            src_c[pl.ds(t * 16, 16)] = sv
            dst_c[pl.ds(t * 16, 16)] = dv
            doff_c[pl.ds(t * 16, 16)] = dv + soff

    def gather(src_c, rows_v, sem):
        pltpu.async_copy(x_hbm.at[src_c], rows_v, sem)

    def drain(src_c, rows_v, sem):
        pltpu.make_async_copy(x_hbm.at[src_c], rows_v, sem).wait()

    def scatter_rows(dst_c, rows_v):
        pltpu.sync_copy(rows_v, agg_s.at[dst_c], add=True)

    def deg_add(doff_c, sem):
        pltpu.async_copy(ones_v, deg_s.at[doff_c], sem)

    def deg_drain(doff_c, sem):
        pltpu.make_async_copy(ones_v, deg_s.at[doff_c], sem).wait()

    def zrows_body(i, carry):
        r = i // (D // 16)
        t = i % (D // 16)
        rows_b[r, pl.ds(t * 16, 16)] = zeros
        return carry
    lax.fori_loop(0, CHUNK * (D // 16), zrows_body, 0)
    base = s * ROWS_PER_TILE
    for b in range(ROWS_PER_TILE // CHUNK):
        pltpu.sync_copy(rows_b, agg_s.at[pl.ds(base + b * CHUNK, CHUNK)])
    rem = ROWS_PER_TILE % CHUNK
    if rem:
        pltpu.sync_copy(
            rows_b.at[pl.ds(0, rem)],
            agg_s.at[pl.ds(base + (ROWS_PER_TILE // CHUNK) * CHUNK, rem)])

    def zbuf_body(t, carry):
        zbuf[pl.ds(t * 16, 16)] = zeros
        return carry
    lax.fori_loop(0, ZBUF // 16, zbuf_body, 0)
    for i in range(DEG_N // ZBUF):
        pltpu.sync_copy(zbuf, deg_s.at[pl.ds(soff + i * ZBUF, ZBUF)])
    drem = DEG_N % ZBUF
    if drem:
        pltpu.sync_copy(zbuf.at[pl.ds(0, drem)],
                        deg_s.at[pl.ds(soff + (DEG_N // ZBUF) * ZBUF, drem)])
    unpack(0, src_a, dst_a, doff_a)
    gather(src_a, rows_a, sem_a)
    deg_add(doff_a, sem_da)
    unpack(1, src_b, dst_b, doff_b)
    gather(src_b, rows_b, sem_b)
    deg_add(doff_b, sem_db)
    plsc.subcore_barrier()

    def pair_body(k, carry):
        drain(src_a, rows_a, sem_a)
        scatter_rows(dst_a, rows_a)
        deg_drain(doff_a, sem_da)
        unpack(2 * k + 2, src_a, dst_a, doff_a)
        gather(src_a, rows_a, sem_a)
        deg_add(doff_a, sem_da)
        drain(src_b, rows_b, sem_b)
        scatter_rows(dst_b, rows_b)

        def prep_b():
            deg_drain(doff_b, sem_db)
            unpack(2 * k + 3, src_b, dst_b, doff_b)
            gather(src_b, rows_b, sem_b)
            deg_add(doff_b, sem_db)
        pl.when(2 * k + 3 <= NCHUNK - 1)(prep_b)
        return carry

    lax.fori_loop(0, (NCHUNK - 1) // 2, pair_body, 0)
    drain(src_a, rows_a, sem_a)
    scatter_rows(dst_a, rows_a)
    deg_drain(doff_a, sem_da)
    deg_drain(doff_b, sem_db)
    plsc.subcore_barrier()

    pltpu.sync_copy(agg_s.at[pl.ds(base, ROWS_PER_TILE)],
                    agg_out.at[c, pl.ds(base, ROWS_PER_TILE)])
    for i in range(DEG_N // ZBUF):
        pltpu.sync_copy(deg_s.at[pl.ds(soff + i * ZBUF, ZBUF)], zbuf)
        pltpu.sync_copy(zbuf, deg_out.at[pl.ds(wid * DEG_N + i * ZBUF, ZBUF)])
    if DEG_N % ZBUF:
        i = DEG_N // ZBUF
        drem = DEG_N % ZBUF
        pltpu.sync_copy(deg_s.at[pl.ds(soff + i * ZBUF, drem)],
                        zbuf.at[pl.ds(0, drem)])
        pltpu.sync_copy(zbuf.at[pl.ds(0, drem)],
                        deg_out.at[pl.ds(wid * DEG_N + i * ZBUF, drem)])


def _sc_agg(x, comb_r):
    mesh = plsc.VectorSubcoreMesh(core_axis_name="c", subcore_axis_name="s")
    fn = functools.partial(
        pl.kernel,
        mesh=mesh,
        out_type=[
            jax.ShapeDtypeStruct((2, N_PAD, D), jnp.float32),
            jax.ShapeDtypeStruct((NW * DEG_N,), jnp.float32),
        ],
        scratch_types=[
            pltpu.VMEM((NCHUNK, CHUNK), jnp.int32),
            pltpu.VMEM((CHUNK,), jnp.int32),
            pltpu.VMEM((CHUNK,), jnp.int32),
            pltpu.VMEM((CHUNK,), jnp.int32),
            pltpu.VMEM((CHUNK,), jnp.int32),
            pltpu.VMEM((CHUNK,), jnp.int32),
            pltpu.VMEM((CHUNK,), jnp.int32),
            pltpu.VMEM((CHUNK, D), jnp.float32),
            pltpu.VMEM((CHUNK, D), jnp.float32),
            pltpu.VMEM((CHUNK,), jnp.float32),
            pltpu.VMEM((ZBUF,), jnp.float32),
            pltpu.VMEM_SHARED((N_PAD, D), jnp.float32),
            pltpu.VMEM_SHARED((16 * DEG_N,), jnp.float32),
            pltpu.SemaphoreType.DMA,
            pltpu.SemaphoreType.DMA,
            pltpu.SemaphoreType.DMA,
            pltpu.SemaphoreType.DMA,
        ],
    )
    return fn(_sc_agg_kernel)(x, comb_r)


def _tc_combine_kernel(x_ref, agg_ref, deg_ref, w_ref, o_ref):
    deg = jnp.sum(deg_ref[...], axis=1)
    den = jnp.maximum(deg, 1.0)
    agg = agg_ref[0] + agg_ref[1]
    nb = agg / den[:, None]
    w = w_ref[...]
    out = lax.dot_general(x_ref[...], w[:, :D], (((1,), (1,)), ((), ())),
                          preferred_element_type=jnp.float32)
    out = out + lax.dot_general(nb, w[:, D:], (((1,), (1,)), ((), ())),
                                preferred_element_type=jnp.float32)
    o_ref[...] = out


def _tc_combine(x, agg_p, deg_p, w):
    blk = 1000
    grid = (N_NODES // blk,)
    return pl.pallas_call(
        _tc_combine_kernel,
        grid=grid,
        in_specs=[
            pl.BlockSpec((blk, D), lambda i: (i, 0)),
            pl.BlockSpec((2, blk, D), lambda i: (0, i, 0)),
            pl.BlockSpec((blk, NW), lambda i: (i, 0)),
            pl.BlockSpec((D, 2 * D), lambda i: (0, 0)),
        ],
        out_specs=pl.BlockSpec((blk, D), lambda i: (i, 0)),
        out_shape=jax.ShapeDtypeStruct((N_NODES, D), jnp.float32),
    )(x, agg_p, deg_p, w)


@jax.jit
def kernel(X, adj, W):
    src = adj[0].astype(jnp.int32)
    dst = adj[1].astype(jnp.int32)
    comb = (src + (dst << 16)).reshape(NW, NCHUNK, CHUNK)
    agg_p, deg_p = _sc_agg(X, comb)
    agg_p = agg_p[:, :N_NODES, :]
    deg_p = deg_p.reshape(NW, DEG_N).T
    return _tc_combine(X, agg_p, deg_p, W)


# speedup vs baseline: 12.8362x; 1.0139x over previous
import functools

import jax
import jax.numpy as jnp
from jax import lax
from jax.experimental import pallas as pl
from jax.experimental.pallas import tpu as pltpu
from jax.experimental.pallas import tpu_sc as plsc

N_NODES = 10000
N_EDGES = 320000
D = 128

NW = 32
CHUNK = 80
NCHUNK = N_EDGES // NW // CHUNK
N_PAD = 10112
ROWS_PER_TILE = N_PAD // 16
DEG_N = N_NODES
ZBUF = 640


def _sc_agg_kernel(x_hbm, comb_hbm,
                   agg_out, deg_out,
                   comb_v, src_a, dst_a, doff_a, src_b, dst_b, doff_b,
                   rows_a, rows_b, ones_v, zbuf,
                   agg_s, deg_s, sem_a, sem_b):
    c = lax.axis_index("c")
    s = lax.axis_index("s")
    wid = c * 16 + s
    soff = s * DEG_N

    pltpu.sync_copy(comb_hbm.at[wid], comb_v)

    ones = jnp.full((16,), 1.0, jnp.float32)
    zeros = jnp.zeros((16,), jnp.float32)
    for t in range(CHUNK // 16):
        ones_v[pl.ds(t * 16, 16)] = ones

    def unpack(j, src_c, dst_c, doff_c):
        for t in range(CHUNK // 16):
            cv = comb_v[j, pl.ds(t * 16, 16)]
            sv = jnp.bitwise_and(cv, 0xFFFF)
            dv = lax.shift_right_logical(cv, 16)
            src_c[pl.ds(t * 16, 16)] = sv
            dst_c[pl.ds(t * 16, 16)] = dv
            doff_c[pl.ds(t * 16, 16)] = dv + soff

    def gather(src_c, rows_v, sem):
        pltpu.async_copy(x_hbm.at[src_c], rows_v, sem)

    def drain(src_c, rows_v, sem):
        pltpu.make_async_copy(x_hbm.at[src_c], rows_v, sem).wait()

    def scatter_rows(dst_c, rows_v):
        pltpu.sync_copy(rows_v, agg_s.at[dst_c], add=True)

    def deg_add(doff_c):
        pltpu.sync_copy(ones_v, deg_s.at[doff_c], add=True)

    def zrows_body(i, carry):
        r = i // (D // 16)
        t = i % (D // 16)
        rows_b[r, pl.ds(t * 16, 16)] = zeros
        return carry
    lax.fori_loop(0, CHUNK * (D // 16), zrows_body, 0)
    base = s * ROWS_PER_TILE
    for b in range(ROWS_PER_TILE // CHUNK):
        pltpu.sync_copy(rows_b, agg_s.at[pl.ds(base + b * CHUNK, CHUNK)])
    rem = ROWS_PER_TILE % CHUNK
    if rem:
        pltpu.sync_copy(
            rows_b.at[pl.ds(0, rem)],
            agg_s.at[pl.ds(base + (ROWS_PER_TILE // CHUNK) * CHUNK, rem)])

    def zbuf_body(t, carry):
        zbuf[pl.ds(t * 16, 16)] = zeros
        return carry
    lax.fori_loop(0, ZBUF // 16, zbuf_body, 0)
    for i in range(DEG_N // ZBUF):
        pltpu.sync_copy(zbuf, deg_s.at[pl.ds(soff + i * ZBUF, ZBUF)])
    drem = DEG_N % ZBUF
    if drem:
        pltpu.sync_copy(zbuf.at[pl.ds(0, drem)],
                        deg_s.at[pl.ds(soff + (DEG_N // ZBUF) * ZBUF, drem)])
    unpack(0, src_a, dst_a, doff_a)
    gather(src_a, rows_a, sem_a)
    deg_add(doff_a)
    plsc.subcore_barrier()

    def pair_body(k, carry):
        unpack(2 * k + 1, src_b, dst_b, doff_b)
        gather(src_b, rows_b, sem_b)
        drain(src_a, rows_a, sem_a)
        scatter_rows(dst_a, rows_a)
        unpack(2 * k + 2, src_a, dst_a, doff_a)
        gather(src_a, rows_a, sem_a)
        deg_add(doff_a)
        drain(src_b, rows_b, sem_b)
        scatter_rows(dst_b, rows_b)
        deg_add(doff_b)
        return carry

    lax.fori_loop(0, (NCHUNK - 1) // 2, pair_body, 0)
    drain(src_a, rows_a, sem_a)
    scatter_rows(dst_a, rows_a)
    plsc.subcore_barrier()

    pltpu.sync_copy(agg_s.at[pl.ds(base, ROWS_PER_TILE)],
                    agg_out.at[c, pl.ds(base, ROWS_PER_TILE)])
    for i in range(DEG_N // ZBUF):
        pltpu.sync_copy(deg_s.at[pl.ds(soff + i * ZBUF, ZBUF)], zbuf)
        pltpu.sync_copy(zbuf, deg_out.at[pl.ds(wid * DEG_N + i * ZBUF, ZBUF)])
    if DEG_N % ZBUF:
        i = DEG_N // ZBUF
        drem = DEG_N % ZBUF
        pltpu.sync_copy(deg_s.at[pl.ds(soff + i * ZBUF, drem)],
                        zbuf.at[pl.ds(0, drem)])
        pltpu.sync_copy(zbuf.at[pl.ds(0, drem)],
                        deg_out.at[pl.ds(wid * DEG_N + i * ZBUF, drem)])


def _sc_agg(x, comb_r):
    mesh = plsc.VectorSubcoreMesh(core_axis_name="c", subcore_axis_name="s")
    fn = functools.partial(
        pl.kernel,
        mesh=mesh,
        out_type=[
            jax.ShapeDtypeStruct((2, N_PAD, D), jnp.float32),
            jax.ShapeDtypeStruct((NW * DEG_N,), jnp.float32),
        ],
        scratch_types=[
            pltpu.VMEM((NCHUNK, CHUNK), jnp.int32),
            pltpu.VMEM((CHUNK,), jnp.int32),
            pltpu.VMEM((CHUNK,), jnp.int32),
            pltpu.VMEM((CHUNK,), jnp.int32),
            pltpu.VMEM((CHUNK,), jnp.int32),
            pltpu.VMEM((CHUNK,), jnp.int32),
            pltpu.VMEM((CHUNK,), jnp.int32),
            pltpu.VMEM((CHUNK, D), jnp.float32),
            pltpu.VMEM((CHUNK, D), jnp.float32),
            pltpu.VMEM((CHUNK,), jnp.float32),
            pltpu.VMEM((ZBUF,), jnp.float32),
            pltpu.VMEM_SHARED((N_PAD, D), jnp.float32),
            pltpu.VMEM_SHARED((16 * DEG_N,), jnp.float32),
            pltpu.SemaphoreType.DMA,
            pltpu.SemaphoreType.DMA,
        ],
    )
    return fn(_sc_agg_kernel)(x, comb_r)


def _tc_combine_kernel(x_ref, agg_ref, deg_ref, w_ref, o_ref):
    deg = jnp.sum(deg_ref[...], axis=1)
    den = jnp.maximum(deg, 1.0)
    agg = agg_ref[0] + agg_ref[1]
    nb = agg / den[:, None]
    w = w_ref[...]
    out = lax.dot_general(x_ref[...], w[:, :D], (((1,), (1,)), ((), ())),
                          preferred_element_type=jnp.float32)
    out = out + lax.dot_general(nb, w[:, D:], (((1,), (1,)), ((), ())),
                                preferred_element_type=jnp.float32)
    o_ref[...] = out


def _tc_combine(x, agg_p, deg_p, w):
    blk = 1000
    grid = (N_NODES // blk,)
    return pl.pallas_call(
        _tc_combine_kernel,
        grid=grid,
        in_specs=[
            pl.BlockSpec((blk, D), lambda i: (i, 0)),
            pl.BlockSpec((2, blk, D), lambda i: (0, i, 0)),
            pl.BlockSpec((blk, NW), lambda i: (i, 0)),
            pl.BlockSpec((D, 2 * D), lambda i: (0, 0)),
        ],
        out_specs=pl.BlockSpec((blk, D), lambda i: (i, 0)),
        out_shape=jax.ShapeDtypeStruct((N_NODES, D), jnp.float32),
    )(x, agg_p, deg_p, w)


@jax.jit
def kernel(X, adj, W):
    src = adj[0].astype(jnp.int32)
    dst = adj[1].astype(jnp.int32)
    comb = (src + (dst << 16)).reshape(NW, NCHUNK, CHUNK)
    agg_p, deg_p = _sc_agg(X, comb)
    deg_p = deg_p.reshape(NW, DEG_N).T
    return _tc_combine(X, agg_p, deg_p, W)
